# baseline (speedup 1.0000x reference)
import functools

import jax
import jax.numpy as jnp
from jax import lax
from jax.experimental import pallas as pl
from jax.experimental.pallas import tpu as pltpu
from jax.experimental.pallas import tpu_sc as plsc

VOCAB = 100000
SEQ = 200
DIM = 64
BATCH = 4096

_NC = 2
_NS = 16
_NW = _NC * _NS
_BT = BATCH // _NW
_EH = DIM // 8


def _tpe_body(x_hbm, tbl_hbm, pos_hbm, out_hbm,
              idx_v, pos_v, rows0, rows1, t0, t1, g0, g1, o0, o1):
    wid = lax.axis_index("s") * _NC + lax.axis_index("c")

    rows = (rows0, rows1)
    tbuf = (t0, t1)
    gsem = (g0, g1)
    osem = (o0, o1)

    pltpu.sync_copy(pos_hbm, pos_v)
    pltpu.sync_copy(x_hbm.at[:, pl.ds(wid * _BT, _BT)], idx_v)

    def gather(s, par):
        return pltpu.make_async_copy(
            tbl_hbm.at[idx_v.at[s]], rows[par], gsem[par])

    def outcopy(s, par):
        return pltpu.make_async_copy(
            tbuf[par], out_hbm.at[s, :, wid], osem[par])

    gather(0, 0).start()
    iota = lax.iota(jnp.int32, 16)

    @pl.loop(0, SEQ, step=2)
    def _seq(sg):
        for par in range(2):
            s = sg + par
            nxt = 1 - par

            @pl.when(s + 1 < SEQ)
            def _():
                gather(s + 1, nxt).start()

            gather(s, par).wait()

            @pl.when(s >= 2)
            def _():
                outcopy(s - 2, par).wait()

            for eh in range(_EH):
                for el in range(8):
                    e = eh * 8 + el
                    pb = plsc.load_gather(
                        pos_v, [jnp.full((16,), s * DIM + e, jnp.int32)])
                    eidx = jnp.full((16,), e, jnp.int32)
                    for g in range(_BT // 16):
                        v = plsc.load_gather(
                            rows[par], [iota + g * 16, eidx])
                        tbuf[par][eh, el, pl.ds(g * 16, 16)] = v + pb

            outcopy(s, par).start()

    outcopy(SEQ - 2, 0).wait()
    outcopy(SEQ - 1, 1).wait()


@jax.jit
def _tpe(xT, token_table, posf):
    f = functools.partial(
        pl.kernel,
        out_type=jax.ShapeDtypeStruct((SEQ, _EH, _NW, 8, _BT), jnp.float32),
        mesh=plsc.VectorSubcoreMesh(core_axis_name="c", subcore_axis_name="s"),
        scratch_types=[
            pltpu.VMEM((SEQ, _BT), jnp.int32),
            pltpu.VMEM((SEQ * DIM,), jnp.float32),
            pltpu.VMEM((_BT, DIM), jnp.float32),
            pltpu.VMEM((_BT, DIM), jnp.float32),
            pltpu.VMEM((_EH, 8, _BT), jnp.float32),
            pltpu.VMEM((_EH, 8, _BT), jnp.float32),
            pltpu.SemaphoreType.DMA,
            pltpu.SemaphoreType.DMA,
            pltpu.SemaphoreType.DMA,
            pltpu.SemaphoreType.DMA,
        ],
        compiler_params=pltpu.CompilerParams(
            use_tc_tiling_on_sc=False, needs_layout_passes=False),
    )(_tpe_body)
    return f(xT, token_table, posf)


def kernel(x, token_table, pos_table):
    xT = x.astype(jnp.int32).T
    posf = pos_table.reshape(-1)
    out = _tpe(xT, token_table, posf)
    return out.transpose(2, 4, 0, 1, 3).reshape(BATCH, SEQ, DIM)


# speedup vs baseline: 1.4492x; 1.4492x over previous
import functools

import jax
import jax.numpy as jnp
from jax import lax
from jax.experimental import pallas as pl
from jax.experimental.pallas import tpu as pltpu
from jax.experimental.pallas import tpu_sc as plsc

VOCAB = 100000
SEQ = 200
DIM = 64
BATCH = 4096

_NC = 2
_NS = 16
_NW = _NC * _NS
_BT = BATCH // _NW
_EH = DIM // 8


def _tpe_body(x_hbm, tbl_hbm, pos_hbm, out_hbm,
              idx_v, pos_v, rows0, rows1, t0, t1, g0, g1, o0, o1):
    wid = lax.axis_index("s") * _NC + lax.axis_index("c")

    rows = (rows0, rows1)
    tbuf = (t0, t1)
    gsem = (g0, g1)
    osem = (o0, o1)

    pltpu.sync_copy(pos_hbm, pos_v)
    pltpu.sync_copy(x_hbm.at[:, pl.ds(wid * _BT, _BT)], idx_v)

    def gather(s, par):
        return pltpu.make_async_copy(
            tbl_hbm.at[idx_v.at[s]], rows[par], gsem[par])

    def outcopy(s, par):
        return pltpu.make_async_copy(
            tbuf[par], out_hbm.at[s, :, wid], osem[par])

    gather(0, 0).start()
    iota = lax.iota(jnp.int32, 16)

    @pl.loop(0, SEQ, step=2)
    def _seq(sg):
        for par in range(2):
            s = sg + par
            nxt = 1 - par

            @pl.when(s + 1 < SEQ)
            def _():
                gather(s + 1, nxt).start()

            gather(s, par).wait()

            @pl.when(s >= 2)
            def _():
                outcopy(s - 2, par).wait()

            for eh in range(_EH):
                for el in range(8):
                    e = eh * 8 + el
                    pb = plsc.load_gather(
                        pos_v, [jnp.full((16,), s * DIM + e, jnp.int32)])
                    eidx = jnp.full((16,), e, jnp.int32)
                    for g in range(_BT // 16):
                        v = plsc.load_gather(
                            rows[par], [iota + g * 16, eidx])
                        tbuf[par][eh, el, pl.ds(g * 16, 16)] = v + pb

            outcopy(s, par).start()

    outcopy(SEQ - 2, 0).wait()
    outcopy(SEQ - 1, 1).wait()


@jax.jit
def _tpe(xT, token_table, posf):
    f = functools.partial(
        pl.kernel,
        out_type=jax.ShapeDtypeStruct((SEQ, _EH, _NW, 8, _BT), jnp.float32),
        mesh=plsc.VectorSubcoreMesh(core_axis_name="c", subcore_axis_name="s"),
        scratch_types=[
            pltpu.VMEM((SEQ, _BT), jnp.int32),
            pltpu.VMEM((SEQ * DIM,), jnp.float32),
            pltpu.VMEM((_BT, DIM + 1), jnp.float32),
            pltpu.VMEM((_BT, DIM + 1), jnp.float32),
            pltpu.VMEM((_EH, 8, _BT), jnp.float32),
            pltpu.VMEM((_EH, 8, _BT), jnp.float32),
            pltpu.SemaphoreType.DMA,
            pltpu.SemaphoreType.DMA,
            pltpu.SemaphoreType.DMA,
            pltpu.SemaphoreType.DMA,
        ],
        compiler_params=pltpu.CompilerParams(
            use_tc_tiling_on_sc=False, needs_layout_passes=False),
    )(_tpe_body)
    return f(xT, token_table, posf)


def kernel(x, token_table, pos_table):
    xT = x.astype(jnp.int32).T
    posf = pos_table.reshape(-1)
    tbl65 = jnp.pad(token_table, ((0, 0), (0, 1)))
    out = _tpe(xT, tbl65, posf)
    return out.transpose(2, 4, 0, 1, 3).reshape(BATCH, SEQ, DIM)


# speedup vs baseline: 1.5354x; 1.0595x over previous
import functools

import jax
import jax.numpy as jnp
from jax import lax
from jax.experimental import pallas as pl
from jax.experimental.pallas import tpu as pltpu
from jax.experimental.pallas import tpu_sc as plsc

VOCAB = 100000
SEQ = 200
DIM = 64
BATCH = 4096

_NC = 2
_NS = 16
_NW = _NC * _NS
_BT = BATCH // _NW
_EH = DIM // 8


def _tpe_body(x_hbm, tbl_hbm, pos_hbm, out_hbm,
              idx_v, pos_v, rows0, rows1, rp, t0, t1, g0, g1, o0, o1):
    wid = lax.axis_index("s") * _NC + lax.axis_index("c")

    rows = (rows0, rows1)
    tbuf = (t0, t1)
    gsem = (g0, g1)
    osem = (o0, o1)

    pltpu.sync_copy(pos_hbm, pos_v)
    pltpu.sync_copy(x_hbm.at[:, pl.ds(wid * _BT, _BT)], idx_v)

    def gather(s, par):
        return pltpu.make_async_copy(
            tbl_hbm.at[idx_v.at[s]], rows[par], gsem[par])

    def outcopy(s, par):
        return pltpu.make_async_copy(
            tbuf[par], out_hbm.at[s, :, wid], osem[par])

    gather(0, 0).start()
    iota = lax.iota(jnp.int32, 16)

    @pl.loop(0, SEQ, step=2)
    def _seq(sg):
        for par in range(2):
            s = sg + par
            nxt = 1 - par

            @pl.when(s + 1 < SEQ)
            def _():
                gather(s + 1, nxt).start()

            gather(s, par).wait()

            @pl.when(s >= 2)
            def _():
                outcopy(s - 2, par).wait()

            for j in range(DIM // 16):
                pj = pos_v[pl.ds(s * DIM + j * 16, 16)]
                for r in range(_BT):
                    rp[r, pl.ds(j * 16, 16)] = (
                        rows[par][r, pl.ds(j * 16, 16)] + pj)

            for eh in range(_EH):
                for el in range(8):
                    eidx = jnp.full((16,), eh * 8 + el, jnp.int32)
                    for g in range(_BT // 16):
                        tbuf[par][eh, el, pl.ds(g * 16, 16)] = (
                            plsc.load_gather(rp, [iota + g * 16, eidx]))

            outcopy(s, par).start()

    outcopy(SEQ - 2, 0).wait()
    outcopy(SEQ - 1, 1).wait()


@jax.jit
def _tpe(xT, token_table, posf):
    f = functools.partial(
        pl.kernel,
        out_type=jax.ShapeDtypeStruct((SEQ, _EH, _NW, 8, _BT), jnp.float32),
        mesh=plsc.VectorSubcoreMesh(core_axis_name="c", subcore_axis_name="s"),
        scratch_types=[
            pltpu.VMEM((SEQ, _BT), jnp.int32),
            pltpu.VMEM((SEQ * DIM,), jnp.float32),
            pltpu.VMEM((_BT, DIM), jnp.float32),
            pltpu.VMEM((_BT, DIM), jnp.float32),
            pltpu.VMEM((_BT, DIM + 1), jnp.float32),
            pltpu.VMEM((_EH, 8, _BT), jnp.float32),
            pltpu.VMEM((_EH, 8, _BT), jnp.float32),
            pltpu.SemaphoreType.DMA,
            pltpu.SemaphoreType.DMA,
            pltpu.SemaphoreType.DMA,
            pltpu.SemaphoreType.DMA,
        ],
        compiler_params=pltpu.CompilerParams(
            use_tc_tiling_on_sc=False, needs_layout_passes=False),
    )(_tpe_body)
    return f(xT, token_table, posf)


def kernel(x, token_table, pos_table):
    xT = x.astype(jnp.int32).T
    posf = pos_table.reshape(-1)
    out = _tpe(xT, token_table, posf)
    return out.transpose(2, 4, 0, 1, 3).reshape(BATCH, SEQ, DIM)


# speedup vs baseline: 3.3817x; 2.2025x over previous
import functools

import jax
import jax.numpy as jnp
from jax import lax
from jax.experimental import pallas as pl
from jax.experimental.pallas import tpu as pltpu
from jax.experimental.pallas import tpu_sc as plsc

VOCAB = 100000
SEQ = 200
DIM = 64
BATCH = 4096

_NC = 2
_NS = 16
_NW = _NC * _NS
_BPW = BATCH // _NW
_H = SEQ // 2
_HP = 56


def _gather_body(x_hbm, tbl_hbm, pos_hbm, out_hbm,
                 idx_v, pos_v, gb0, gb1, ob0, ob1, g0, g1, o0, o1):
    wid = lax.axis_index("s") * _NC + lax.axis_index("c")
    b0 = wid * _BPW

    gbuf = (gb0, gb1)
    obuf = (ob0, ob1)
    gsem = (g0, g1)
    osem = (o0, o1)

    pltpu.sync_copy(pos_hbm, pos_v)
    pltpu.sync_copy(x_hbm.at[pl.ds(b0, _BPW)], idx_v)

    def gather(i, par):
        return [pltpu.make_async_copy(
            tbl_hbm.at[idx_v.at[i, k]], gbuf[par].at[k], gsem[par])
            for k in range(2)]

    def outcopy(i, par):
        return pltpu.make_async_copy(
            obuf[par], out_hbm.at[b0 + i, :, pl.ds(0, _H // 2)], osem[par])

    for cp in gather(0, 0):
        cp.start()

    @pl.loop(0, _BPW, step=2)
    def _row(g):
        for par in range(2):
            i = g + par
            nxt = 1 - par

            @pl.when(i + 1 < _BPW)
            def _():
                for cp in gather(i + 1, nxt):
                    cp.start()

            for cp in gather(i, par):
                cp.wait()

            @pl.when(i >= 2)
            def _():
                outcopy(i - 2, par).wait()

            @pl.loop(0, _H // 2)
            def _pos(u):
                for k in range(2):
                    for h in range(2):
                        for j in range(DIM // 16):
                            sl = pl.ds(j * 16, 16)
                            obuf[par][k, u, pl.ds(h * DIM + j * 16, 16)] = (
                                gbuf[par][k, 2 * u + h, sl]
                                + pos_v[k, 2 * u + h, sl])

            outcopy(i, par).start()

    outcopy(_BPW - 2, 0).wait()
    outcopy(_BPW - 1, 1).wait()


def _transpose_body(x_ref, o_ref):
    def body(u, carry):
        for k in range(2):
            xm = x_ref[:, k, u, :]
            y = xm.T.reshape(2, 8, 8, 128)
            o_ref[pl.ds(k * SEQ // 2 + 2 * u, 2), :, 0, :, :] = y
        return carry

    lax.fori_loop(0, _H // 2, body, 0)


@jax.jit
def _tpe(x3, token_table, pos3):
    sc = functools.partial(
        pl.kernel,
        out_type=jax.ShapeDtypeStruct((BATCH, 2, _HP, 128), jnp.float32),
        mesh=plsc.VectorSubcoreMesh(core_axis_name="c", subcore_axis_name="s"),
        scratch_types=[
            pltpu.VMEM((_BPW, 2, _H), jnp.int32),
            pltpu.VMEM((2, _H, DIM), jnp.float32),
            pltpu.VMEM((2, _H, DIM), jnp.float32),
            pltpu.VMEM((2, _H, DIM), jnp.float32),
            pltpu.VMEM((2, _H // 2, 128), jnp.float32),
            pltpu.VMEM((2, _H // 2, 128), jnp.float32),
            pltpu.SemaphoreType.DMA,
            pltpu.SemaphoreType.DMA,
            pltpu.SemaphoreType.DMA,
            pltpu.SemaphoreType.DMA,
        ],
        compiler_params=pltpu.CompilerParams(use_tc_tiling_on_sc=False),
    )(_gather_body)
    lin = sc(x3, token_table, pos3)

    tc = pl.pallas_call(
        _transpose_body,
        out_shape=jax.ShapeDtypeStruct((SEQ, 8, _NW, 8, 128), jnp.float32),
        grid=(_NW,),
        in_specs=[pl.BlockSpec((_BPW, 2, _HP, 128), lambda w: (w, 0, 0, 0))],
        out_specs=pl.BlockSpec((SEQ, 8, 1, 8, 128), lambda w: (0, 0, w, 0, 0)),
        compiler_params=pltpu.CompilerParams(
            dimension_semantics=("arbitrary",)),
    )
    return tc(lin)


def kernel(x, token_table, pos_table):
    x3 = x.reshape(BATCH, 2, _H).astype(jnp.int32)
    pos3 = pos_table.reshape(2, _H, DIM)
    out = _tpe(x3, token_table, pos3)
    return out.transpose(2, 4, 0, 1, 3).reshape(BATCH, SEQ, DIM)
